# baseline (speedup 1.0000x reference)
import functools

import jax
import jax.numpy as jnp
from jax import lax
from jax.experimental import pallas as pl
from jax.experimental.pallas import tpu as pltpu
from jax.experimental.pallas import tpu_sc as plsc

_CHUNK = 128
_PAD = 128


@functools.lru_cache(maxsize=None)
def _make_sc_gather(V, D, B, NC, NS):
    NW = NC * NS
    b_per_w = B // NW
    n_ch = b_per_w // _CHUNK
    mesh = plsc.VectorSubcoreMesh(core_axis_name="c", subcore_axis_name="s")

    @functools.partial(
        pl.kernel,
        mesh=mesh,
        out_type=jax.ShapeDtypeStruct((B, _PAD), jnp.float32),
        scratch_types=[
            pltpu.VMEM((b_per_w,), jnp.int32),
            pltpu.VMEM((b_per_w, D), jnp.float32),
            pltpu.SemaphoreType.DMA,
            pltpu.SemaphoreType.DMA,
        ],
        compiler_params=pltpu.CompilerParams(use_tc_tiling_on_sc=False),
    )
    def k(idx_hbm, table_hbm, out_hbm, idx_v, rows_v, sem, sem_i):
        wid = lax.axis_index("s") * NC + lax.axis_index("c")
        base = wid * b_per_w
        half = b_per_w // 2
        idx_copies = [
            pltpu.async_copy(
                idx_hbm.at[pl.ds(base + h * half, half)],
                idx_v.at[pl.ds(h * half, half)],
                sem_i,
            )
            for h in range(2)
        ]
        gathers = []
        for h in range(2):
            idx_copies[h].wait()
            gathers.append(
                pltpu.async_copy(
                    table_hbm.at[idx_v.at[pl.ds(h * half, half)]],
                    rows_v.at[pl.ds(h * half, half)],
                    sem,
                )
            )
        writes = []
        for h in range(2):
            gathers[h].wait()
            writes.append(
                pltpu.async_copy(
                    rows_v.at[pl.ds(h * half, half)],
                    out_hbm.at[pl.ds(base + h * half, half), pl.ds(0, D)],
                    sem_i,
                )
            )
        for w in writes:
            w.wait()

    return k


def kernel(domain_ids, table):
    (B,) = domain_ids.shape
    V, D = table.shape
    info = plsc.get_sparse_core_info()
    out = _make_sc_gather(V, D, B, info.num_cores, info.num_subcores)(
        domain_ids.astype(jnp.int32), table
    )
    return out[:, :D]


# speedup vs baseline: 1.0119x; 1.0119x over previous
import functools

import jax
import jax.numpy as jnp
from jax import lax
from jax.experimental import pallas as pl
from jax.experimental.pallas import tpu as pltpu
from jax.experimental.pallas import tpu_sc as plsc

_CHUNK = 128
_PAD = 128


@functools.lru_cache(maxsize=None)
def _make_sc_gather(V, D, B, NC, NS):
    NW = NC * NS
    b_per_w = B // NW
    n_ch = b_per_w // _CHUNK
    mesh = plsc.VectorSubcoreMesh(core_axis_name="c", subcore_axis_name="s")

    @functools.partial(
        pl.kernel,
        mesh=mesh,
        out_type=jax.ShapeDtypeStruct((B, _PAD), jnp.float32),
        scratch_types=[
            pltpu.VMEM((b_per_w,), jnp.int32),
            pltpu.VMEM((b_per_w, D), jnp.float32),
            pltpu.SemaphoreType.DMA,
            pltpu.SemaphoreType.DMA,
        ],
        compiler_params=pltpu.CompilerParams(
            use_tc_tiling_on_sc=False, allow_input_fusion=[True, True]
        ),
    )
    def k(idx_hbm, table_hbm, out_hbm, idx_v, rows_v, sem, sem_i):
        wid = lax.axis_index("s") * NC + lax.axis_index("c")
        base = wid * b_per_w
        half = b_per_w // 2
        idx_copies = [
            pltpu.async_copy(
                idx_hbm.at[pl.ds(base + h * half, half)],
                idx_v.at[pl.ds(h * half, half)],
                sem_i,
            )
            for h in range(2)
        ]
        gathers = []
        for h in range(2):
            idx_copies[h].wait()
            gathers.append(
                pltpu.async_copy(
                    table_hbm.at[idx_v.at[pl.ds(h * half, half)]],
                    rows_v.at[pl.ds(h * half, half)],
                    sem,
                )
            )
        for g in gathers:
            g.wait()
        pltpu.sync_copy(
            rows_v,
            out_hbm.at[pl.ds(base, b_per_w), pl.ds(0, D)],
        )

    return k


def kernel(domain_ids, table):
    (B,) = domain_ids.shape
    V, D = table.shape
    info = plsc.get_sparse_core_info()
    out = _make_sc_gather(V, D, B, info.num_cores, info.num_subcores)(
        domain_ids.astype(jnp.int32), table
    )
    return out[:, :D]
